# baseline (speedup 1.0000x reference)
import functools

import jax
import jax.numpy as jnp
from jax import lax
from jax.experimental import pallas as pl
from jax.experimental.pallas import tpu as pltpu
from jax.experimental.pallas import tpu_sc as plsc

N_ROWS = 1048576
NC, NS = 2, 16
NW = NC * NS
ROWS_PER_W = N_ROWS // NW
CHUNK_ROWS = 8192
N_CHUNKS = ROWS_PER_W // CHUNK_ROWS
IN_W = CHUNK_ROWS * 3
OUT_W = CHUNK_ROWS * 4
NV = OUT_W // 16


def _sc_body(x_hbm, w_hbm, out_hbm, in_buf, out_buf, w_buf):
    cid = lax.axis_index("c")
    sid = lax.axis_index("s")
    wid = sid * NC + cid

    i = lax.iota(jnp.int32, 16)
    c = i & 3
    r = i >> 2
    s = (c & 1) * (1 + ((c >> 1) & 1))
    base_idx = r * 3 + s

    pltpu.sync_copy(w_hbm, w_buf)
    wv = jnp.where(c == 0, jnp.float32(0.0), plsc.load_gather(w_buf, [s]))

    in_base = wid * (ROWS_PER_W * 3)
    out_base = wid * (ROWS_PER_W * 4)
    for k in range(N_CHUNKS):
        pltpu.sync_copy(x_hbm.at[pl.ds(in_base + k * IN_W, IN_W)], in_buf)

        @plsc.parallel_loop(0, NV, unroll=8)
        def _(j):
            idx = base_idx + j * 12
            vals = plsc.load_gather(in_buf, [idx])
            out_buf[pl.ds(j * 16, 16)] = vals * wv

        pltpu.sync_copy(out_buf, out_hbm.at[pl.ds(out_base + k * OUT_W, OUT_W)])


@jax.jit
def _sc_spmm(x_flat, w16):
    mesh = plsc.VectorSubcoreMesh(core_axis_name="c", subcore_axis_name="s")
    f = pl.kernel(
        _sc_body,
        out_type=jax.ShapeDtypeStruct((N_ROWS * 4,), jnp.float32),
        mesh=mesh,
        scratch_types=[
            pltpu.VMEM((IN_W,), jnp.float32),
            pltpu.VMEM((OUT_W,), jnp.float32),
            pltpu.VMEM((16,), jnp.float32),
        ],
        compiler_params=pltpu.CompilerParams(needs_layout_passes=False),
    )
    return f(x_flat, w16)


def kernel(x, weights):
    x_flat = x.reshape(-1)
    w16 = jnp.pad(weights.astype(jnp.float32), (0, 13))
    out_flat = _sc_spmm(x_flat, w16)
    return out_flat.reshape(N_ROWS, 4)
